# baseline (speedup 1.0000x reference)
import functools

import jax
import jax.numpy as jnp
from jax import lax
from jax.experimental import pallas as pl
from jax.experimental.pallas import tpu as pltpu
from jax.experimental.pallas import tpu_sc as plsc

_NC = 2
_NS = 16
_NW = _NC * _NS
_L = 16
_K = 64
_RND = 4


def _make_sc_quantize(n: int, c: int, h: int, w: int):
    rpw = (n * c) // _NW
    rpr = rpw // _RND
    mesh = plsc.VectorSubcoreMesh(
        core_axis_name="c", subcore_axis_name="s",
        num_cores=_NC, num_subcores=_NS)

    @functools.partial(
        pl.kernel,
        out_type=(
            jax.ShapeDtypeStruct((n, c, h, w), jnp.float32),
            jax.ShapeDtypeStruct((n, c, h, w), jnp.int32),
        ),
        mesh=mesh,
        compiler_params=pltpu.CompilerParams(needs_layout_passes=False),
        scratch_types=[
            pltpu.VMEM((rpr * h, w), jnp.float32),
            pltpu.VMEM((rpr * h, w), jnp.float32),
            pltpu.VMEM((rpr * h, w), jnp.int32),
            pltpu.VMEM((_K,), jnp.float32),
            pltpu.VMEM((_K,), jnp.float32),
        ],
    )
    def qk(x_hbm, ctr_hbm, hard_hbm, idx_hbm, xv, hv, iv, cv, bv):
        wid = lax.axis_index("s") * _NC + lax.axis_index("c")
        wpn = c // rpw
        nb = wid // wpn
        cb = (wid % wpn) * rpw

        pltpu.sync_copy(ctr_hbm, cv)

        lane = lax.iota(jnp.int32, _L)
        for k in range(_K // _L):
            j = lane + (k * _L)
            c0 = plsc.load_gather(cv, [j])
            c1 = plsc.load_gather(cv, [jnp.minimum(j + 1, _K - 1)])
            mid = (c0 + c1) * 0.5
            bv[pl.ds(k * _L, _L)] = jnp.where(j == _K - 1, jnp.inf, mid)

        def _bcast(j):
            return plsc.load_gather(bv, [jnp.full((_L,), j, jnp.int32)])
        b7, b15, b23, b31 = _bcast(7), _bcast(15), _bcast(23), _bcast(31)
        b39, b47, b55 = _bcast(39), _bcast(47), _bcast(55)

        def search(xs):
            m1 = b31 < xs
            pos = jnp.where(m1, 32, 0)
            m2 = jnp.where(m1, b47, b15) < xs
            pos = jnp.where(m2, pos + 16, pos)
            m3 = jnp.where(m2, jnp.where(m1, b55, b23),
                           jnp.where(m1, b39, b7)) < xs
            pos = jnp.where(m3, pos + 8, pos)
            for s in (4, 2, 1):
                m = plsc.load_gather(bv, [pos + (s - 1)])
                pos = jnp.where(m < xs, pos + s, pos)
            return pos

        for rnd in range(_RND):
            cs = cb + rnd * rpr
            pltpu.sync_copy(
                x_hbm.at[nb, pl.ds(cs, rpr)].reshape(rpr * h, w), xv)

            @plsc.parallel_loop(0, rpr * h, unroll=6)
            def _(i):
                for off in (0, w - _L):
                    xs = xv[i, pl.ds(off, _L)]
                    pos = search(xs)
                    hv[i, pl.ds(off, _L)] = plsc.load_gather(cv, [pos])
                    iv[i, pl.ds(off, _L)] = pos

            pltpu.sync_copy(
                hv, hard_hbm.at[nb, pl.ds(cs, rpr)].reshape(rpr * h, w))
            pltpu.sync_copy(
                iv, idx_hbm.at[nb, pl.ds(cs, rpr)].reshape(rpr * h, w))

    return qk


def kernel(x, centers):
    n, c, h, w = x.shape
    assert (n * c) % (_NW * _RND) == 0 and w >= _L
    hard, idx = _make_sc_quantize(n, c, h, w)(x, centers)
    return (hard, hard, idx)


# speedup vs baseline: 1.1654x; 1.1654x over previous
import functools

import jax
import jax.numpy as jnp
from jax import lax
from jax.experimental import pallas as pl
from jax.experimental.pallas import tpu as pltpu
from jax.experimental.pallas import tpu_sc as plsc

_NC = 2
_NS = 16
_NW = _NC * _NS
_L = 16
_K = 64
_RND = 6


def _make_sc_quantize(n: int, c: int, h: int, w: int):
    rpw = (n * c) // _NW
    rpr = rpw // _RND
    mesh = plsc.VectorSubcoreMesh(
        core_axis_name="c", subcore_axis_name="s",
        num_cores=_NC, num_subcores=_NS)

    @functools.partial(
        pl.kernel,
        out_type=(
            jax.ShapeDtypeStruct((n, c, h, w), jnp.float32),
            jax.ShapeDtypeStruct((n, c, h, w), jnp.int32),
        ),
        mesh=mesh,
        compiler_params=pltpu.CompilerParams(needs_layout_passes=False),
        scratch_types=[
            pltpu.VMEM((2, rpr * h, w), jnp.float32),
            pltpu.VMEM((2, rpr * h, w), jnp.int32),
            pltpu.VMEM((_K,), jnp.float32),
            pltpu.VMEM((_K,), jnp.float32),
            pltpu.SemaphoreType.DMA,
            pltpu.SemaphoreType.DMA,
            pltpu.SemaphoreType.DMA,
            pltpu.SemaphoreType.DMA,
        ],
    )
    def qk(x_hbm, ctr_hbm, hard_hbm, idx_hbm,
           xv, iv, cv, bv, si0, si1, so0, so1):
        wid = lax.axis_index("s") * _NC + lax.axis_index("c")
        wpn = c // rpw
        nb = wid // wpn
        cb = (wid % wpn) * rpw
        sin = (si0, si1)
        sout = (so0, so1)

        pltpu.sync_copy(ctr_hbm, cv)

        lane = lax.iota(jnp.int32, _L)
        for k in range(_K // _L):
            j = lane + (k * _L)
            c0 = plsc.load_gather(cv, [j])
            c1 = plsc.load_gather(cv, [jnp.minimum(j + 1, _K - 1)])
            mid = (c0 + c1) * 0.5
            bv[pl.ds(k * _L, _L)] = jnp.where(j == _K - 1, jnp.inf, mid)

        def _bcast(j):
            return plsc.load_gather(bv, [jnp.full((_L,), j, jnp.int32)])
        b7, b15, b23, b31 = _bcast(7), _bcast(15), _bcast(23), _bcast(31)
        b39, b47, b55 = _bcast(39), _bcast(47), _bcast(55)

        def search(xs):
            m1 = b31 < xs
            pos = jnp.where(m1, 32, 0)
            m2 = jnp.where(m1, b47, b15) < xs
            pos = jnp.where(m2, pos + 16, pos)
            m3 = jnp.where(m2, jnp.where(m1, b55, b23),
                           jnp.where(m1, b39, b7)) < xs
            pos = jnp.where(m3, pos + 8, pos)
            for s in (4, 2, 1):
                m = plsc.load_gather(bv, [pos + (s - 1)])
                pos = jnp.where(m < xs, pos + s, pos)
            return pos

        def hbm_block(ref, rnd):
            cs = cb + rnd * rpr
            return ref.at[nb, pl.ds(cs, rpr)].reshape(rpr * h, w)

        def start_in(rnd):
            p = rnd % 2
            return pltpu.async_copy(hbm_block(x_hbm, rnd), xv.at[p], sin[p])

        pend_in = start_in(0)
        pend_out = None
        for rnd in range(_RND):
            p = rnd % 2
            pend_in.wait()
            if rnd + 1 < _RND:
                if pend_out is not None:
                    for d in pend_out:
                        d.wait()
                    pend_out = None
                pend_in = start_in(rnd + 1)

            @plsc.parallel_loop(0, rpr * h, unroll=4)
            def _(i):
                xa = xv[p, i, pl.ds(0, _L)]
                xb = xv[p, i, pl.ds(w - _L, _L)]
                pa = search(xa)
                pb = search(xb)
                xv[p, i, pl.ds(0, _L)] = plsc.load_gather(cv, [pa])
                iv[p, i, pl.ds(0, _L)] = pa
                xv[p, i, pl.ds(w - _L, _L)] = plsc.load_gather(cv, [pb])
                iv[p, i, pl.ds(w - _L, _L)] = pb

            prev = pend_out
            pend_out = (
                pltpu.async_copy(xv.at[p], hbm_block(hard_hbm, rnd), sout[p]),
                pltpu.async_copy(iv.at[p], hbm_block(idx_hbm, rnd), sout[p]),
            )
            if prev is not None:
                for d in prev:
                    d.wait()
        for d in pend_out:
            d.wait()

    return qk


def kernel(x, centers):
    n, c, h, w = x.shape
    assert (n * c) % (_NW * _RND) == 0 and w >= _L
    hard, idx = _make_sc_quantize(n, c, h, w)(x, centers)
    return (hard, hard, idx)


# speedup vs baseline: 1.2442x; 1.0676x over previous
import functools

import jax
import jax.numpy as jnp
from jax import lax
from jax.experimental import pallas as pl
from jax.experimental.pallas import tpu as pltpu
from jax.experimental.pallas import tpu_sc as plsc

_NC = 2
_NS = 16
_NW = _NC * _NS
_L = 16
_K = 64
_RND = 8
_NBUF = 3


def _make_sc_quantize(n: int, c: int, h: int, w: int):
    rpw = (n * c) // _NW
    rpr = rpw // _RND
    mesh = plsc.VectorSubcoreMesh(
        core_axis_name="c", subcore_axis_name="s",
        num_cores=_NC, num_subcores=_NS)

    @functools.partial(
        pl.kernel,
        out_type=(
            jax.ShapeDtypeStruct((n, c, h, w), jnp.float32),
            jax.ShapeDtypeStruct((n, c, h, w), jnp.int32),
        ),
        mesh=mesh,
        compiler_params=pltpu.CompilerParams(needs_layout_passes=False),
        scratch_types=[
            pltpu.VMEM((_NBUF, rpr * h, w), jnp.float32),
            pltpu.VMEM((_NBUF, rpr * h, w), jnp.int32),
            pltpu.VMEM((_K,), jnp.float32),
            pltpu.VMEM((_K,), jnp.float32),
            pltpu.SemaphoreType.DMA,
            pltpu.SemaphoreType.DMA,
            pltpu.SemaphoreType.DMA,
            pltpu.SemaphoreType.DMA,
            pltpu.SemaphoreType.DMA,
            pltpu.SemaphoreType.DMA,
        ],
    )
    def qk(x_hbm, ctr_hbm, hard_hbm, idx_hbm,
           xv, iv, cv, bv, si0, si1, si2, so0, so1, so2):
        wid = lax.axis_index("s") * _NC + lax.axis_index("c")
        wpn = c // rpw
        nb = wid // wpn
        cb = (wid % wpn) * rpw
        sin = (si0, si1, si2)
        sout = (so0, so1, so2)

        pltpu.sync_copy(ctr_hbm, cv)

        lane = lax.iota(jnp.int32, _L)
        for k in range(_K // _L):
            j = lane + (k * _L)
            c0 = plsc.load_gather(cv, [j])
            c1 = plsc.load_gather(cv, [jnp.minimum(j + 1, _K - 1)])
            mid = (c0 + c1) * 0.5
            bv[pl.ds(k * _L, _L)] = jnp.where(j == _K - 1, jnp.inf, mid)

        def _bcast(j):
            return plsc.load_gather(bv, [jnp.full((_L,), j, jnp.int32)])
        b7, b15, b23, b31 = _bcast(7), _bcast(15), _bcast(23), _bcast(31)
        b39, b47, b55 = _bcast(39), _bcast(47), _bcast(55)

        def search(xs):
            m1 = b31 < xs
            pos = jnp.where(m1, 32, 0)
            m2 = jnp.where(m1, b47, b15) < xs
            pos = jnp.where(m2, pos + 16, pos)
            m3 = jnp.where(m2, jnp.where(m1, b55, b23),
                           jnp.where(m1, b39, b7)) < xs
            pos = jnp.where(m3, pos + 8, pos)
            for s in (4, 2, 1):
                m = plsc.load_gather(bv, [pos + (s - 1)])
                pos = jnp.where(m < xs, pos + s, pos)
            return pos

        def hbm_block(ref, rnd):
            cs = cb + rnd * rpr
            return ref.at[nb, pl.ds(cs, rpr)].reshape(rpr * h, w)

        def start_in(rnd):
            p = rnd % _NBUF
            return pltpu.async_copy(hbm_block(x_hbm, rnd), xv.at[p], sin[p])

        pend_in = {0: start_in(0)}
        pend_out = {}
        for rnd in range(_RND):
            p = rnd % _NBUF
            pend_in.pop(rnd).wait()
            if rnd + 1 < _RND:
                for d in pend_out.pop(rnd - 2, ()):
                    d.wait()
                pend_in[rnd + 1] = start_in(rnd + 1)

            @plsc.parallel_loop(0, rpr * h, unroll=4)
            def _(i):
                xa = xv[p, i, pl.ds(0, _L)]
                xb = xv[p, i, pl.ds(w - _L, _L)]
                pa = search(xa)
                pb = search(xb)
                xv[p, i, pl.ds(0, _L)] = plsc.load_gather(cv, [pa])
                iv[p, i, pl.ds(0, _L)] = pa
                xv[p, i, pl.ds(w - _L, _L)] = plsc.load_gather(cv, [pb])
                iv[p, i, pl.ds(w - _L, _L)] = pb

            pend_out[rnd] = (
                pltpu.async_copy(xv.at[p], hbm_block(hard_hbm, rnd), sout[p]),
                pltpu.async_copy(iv.at[p], hbm_block(idx_hbm, rnd), sout[p]),
            )
        for ds in pend_out.values():
            for d in ds:
                d.wait()

    return qk


def kernel(x, centers):
    n, c, h, w = x.shape
    assert (n * c) % (_NW * _RND) == 0 and w >= _L
    hard, idx = _make_sc_quantize(n, c, h, w)(x, centers)
    return (hard, hard, idx)
